# baseline (speedup 1.0000x reference)
import functools

import jax
import jax.numpy as jnp
from jax import lax
from jax.experimental import pallas as pl
from jax.experimental.pallas import tpu as pltpu
from jax.experimental.pallas import tpu_sc as plsc

NUM_BINS = 32
EMBED_DIM = 64
R = 512
L = 16


@functools.lru_cache(maxsize=None)
def _sc_lookup(n_total: int):
    info = plsc.get_sparse_core_info()
    nc, ns = info.num_cores, info.num_subcores
    nw = nc * ns
    per_w = n_total // nw
    assert per_w * nw == n_total and per_w % (2 * R) == 0
    n_groups = per_w // R
    mesh = plsc.VectorSubcoreMesh(core_axis_name="c", subcore_axis_name="s")

    scratch = [
        pltpu.VMEM((per_w,), jnp.int32),
        pltpu.VMEM((NUM_BINS, EMBED_DIM), jnp.float32),
        pltpu.VMEM((R, EMBED_DIM), jnp.float32),
        pltpu.VMEM((R, EMBED_DIM), jnp.float32),
        pltpu.SemaphoreType.DMA,
        pltpu.SemaphoreType.DMA,
    ]

    @functools.partial(
        pl.kernel,
        out_type=jax.ShapeDtypeStruct((n_total, EMBED_DIM), jnp.float32),
        mesh=mesh,
        scratch_types=scratch,
        compiler_params=pltpu.CompilerParams(
            use_tc_tiling_on_sc=False, needs_layout_passes=False),
    )
    def k(idx_hbm, table_hbm, out_hbm, idx_v, table_v, rows0, rows1, s0, s1):
        wid = lax.axis_index("s") * nc + lax.axis_index("c")
        base = wid * per_w
        pltpu.sync_copy(table_hbm, table_v)
        pltpu.sync_copy(idx_hbm.at[pl.ds(base, per_w)], idx_v)

        lane = lax.iota(jnp.int32, L)
        colmod = [(lane + c) & (L - 1) for c in range(L)]

        BATCH = 16

        def compute_group(g, rows_ref):
            @plsc.parallel_loop(0, R // L, unroll=2)
            def blk(i):
                bins = idx_v[pl.ds(g * R + i * L, L)]
                rowv = i * L + lane
                for cb in range(0, EMBED_DIM, L):
                    for c0 in range(0, L, BATCH):
                        colvs = [colmod[c0 + c] + cb for c in range(BATCH)]
                        vs = [plsc.load_gather(table_v, [bins, cv])
                              for cv in colvs]
                        for cv, v in zip(colvs, vs):
                            plsc.store_scatter(rows_ref, [rowv, cv], v)

        def fire_store(g, rows_ref, sem):
            return pltpu.async_copy(
                rows_ref, out_hbm.at[pl.ds(base + g * R, R)], sem)

        def wait_store(g, rows_ref, sem):
            pltpu.make_async_copy(
                rows_ref, out_hbm.at[pl.ds(base + g * R, R)], sem).wait()

        compute_group(0, rows0)
        fire_store(0, rows0, s0)
        compute_group(1, rows1)
        fire_store(1, rows1, s1)

        def body(gh, carry):
            g0 = gh * 2
            wait_store(g0 - 2, rows0, s0)
            compute_group(g0, rows0)
            fire_store(g0, rows0, s0)
            wait_store(g0 - 1, rows1, s1)
            compute_group(g0 + 1, rows1)
            fire_store(g0 + 1, rows1, s1)
            return carry

        lax.fori_loop(1, n_groups // 2, body, 0)
        wait_store(n_groups - 2, rows0, s0)
        wait_store(n_groups - 1, rows1, s1)

    return k


def kernel(velocity_bins, table):
    b, s = velocity_bins.shape
    n = b * s
    idx = velocity_bins.astype(jnp.int32).reshape(n)
    out = _sc_lookup(n)(idx, table)
    return out.reshape(b, s, EMBED_DIM)


# speedup vs baseline: 1.2195x; 1.2195x over previous
import functools

import jax
import jax.numpy as jnp
from jax import lax
from jax.experimental import pallas as pl
from jax.experimental.pallas import tpu as pltpu
from jax.experimental.pallas import tpu_sc as plsc

NUM_BINS = 32
EMBED_DIM = 64
R = 512
L = 16


@functools.lru_cache(maxsize=None)
def _sc_lookup(n_total: int):
    info = plsc.get_sparse_core_info()
    nc, ns = info.num_cores, info.num_subcores
    nw = nc * ns
    per_w = n_total // nw
    assert per_w * nw == n_total and per_w % (2 * R) == 0
    n_groups = per_w // R
    mesh = plsc.VectorSubcoreMesh(core_axis_name="c", subcore_axis_name="s")

    scratch = [
        pltpu.VMEM((per_w,), jnp.int32),
        pltpu.VMEM((NUM_BINS, EMBED_DIM), jnp.float32),
        pltpu.VMEM((R, EMBED_DIM), jnp.float32),
        pltpu.VMEM((R, EMBED_DIM), jnp.float32),
        pltpu.SemaphoreType.DMA,
        pltpu.SemaphoreType.DMA,
    ]

    @functools.partial(
        pl.kernel,
        out_type=jax.ShapeDtypeStruct((n_total, EMBED_DIM), jnp.float32),
        mesh=mesh,
        scratch_types=scratch,
        compiler_params=pltpu.CompilerParams(
            use_tc_tiling_on_sc=False, needs_layout_passes=False),
    )
    def k(idx_hbm, table_hbm, out_hbm, idx_v, table_v, rows0, rows1, s0, s1):
        wid = lax.axis_index("s") * nc + lax.axis_index("c")
        base = wid * per_w
        pltpu.sync_copy(table_hbm, table_v)
        pltpu.sync_copy(idx_hbm.at[pl.ds(base, per_w)], idx_v)

        lane = lax.iota(jnp.int32, L)
        colmod = [(lane + c) & (L - 1) for c in range(L)]

        BATCH = 8

        def compute_group(g, rows_ref):
            @plsc.parallel_loop(0, R // L, unroll=2)
            def blk(i):
                bins = idx_v[pl.ds(g * R + i * L, L)]
                rowv = i * L + lane
                for cb in range(0, EMBED_DIM, L):
                    for c0 in range(0, L, BATCH):
                        colvs = [colmod[c0 + c] + cb for c in range(BATCH)]
                        vs = [plsc.load_gather(table_v, [bins, cv])
                              for cv in colvs]
                        for cv, v in zip(colvs, vs):
                            plsc.store_scatter(rows_ref, [rowv, cv], v)

        def fire_store(g, rows_ref, sem):
            return pltpu.async_copy(
                rows_ref, out_hbm.at[pl.ds(base + g * R, R)], sem)

        def wait_store(g, rows_ref, sem):
            pltpu.make_async_copy(
                rows_ref, out_hbm.at[pl.ds(base + g * R, R)], sem).wait()

        compute_group(0, rows0)
        fire_store(0, rows0, s0)
        compute_group(1, rows1)
        fire_store(1, rows1, s1)

        def body(gh, carry):
            g0 = gh * 2
            wait_store(g0 - 2, rows0, s0)
            compute_group(g0, rows0)
            fire_store(g0, rows0, s0)
            wait_store(g0 - 1, rows1, s1)
            compute_group(g0 + 1, rows1)
            fire_store(g0 + 1, rows1, s1)
            return carry

        lax.fori_loop(1, n_groups // 2, body, 0)
        wait_store(n_groups - 2, rows0, s0)
        wait_store(n_groups - 1, rows1, s1)

    return k


def kernel(velocity_bins, table):
    b, s = velocity_bins.shape
    n = b * s
    idx = velocity_bins.astype(jnp.int32).reshape(n)
    out = _sc_lookup(n)(idx, table)
    return out.reshape(b, s, EMBED_DIM)


# speedup vs baseline: 1.5805x; 1.2960x over previous
import functools

import jax
import jax.numpy as jnp
from jax import lax
from jax.experimental import pallas as pl
from jax.experimental.pallas import tpu as pltpu
from jax.experimental.pallas import tpu_sc as plsc

NUM_BINS = 32
EMBED_DIM = 64
R = 512
L = 16


@functools.lru_cache(maxsize=None)
def _sc_lookup(n_total: int):
    info = plsc.get_sparse_core_info()
    nc, ns = info.num_cores, info.num_subcores
    nw = nc * ns
    per_w = n_total // nw
    assert per_w * nw == n_total and per_w % (2 * R) == 0
    n_groups = per_w // R
    mesh = plsc.VectorSubcoreMesh(core_axis_name="c", subcore_axis_name="s")

    scratch = [
        pltpu.VMEM((per_w,), jnp.int32),
        pltpu.VMEM((NUM_BINS, EMBED_DIM), jnp.float32),
        pltpu.VMEM((R, EMBED_DIM), jnp.float32),
        pltpu.VMEM((R, EMBED_DIM), jnp.float32),
        pltpu.SemaphoreType.DMA,
        pltpu.SemaphoreType.DMA,
    ]

    @functools.partial(
        pl.kernel,
        out_type=jax.ShapeDtypeStruct((n_total, EMBED_DIM), jnp.float32),
        mesh=mesh,
        scratch_types=scratch,
        compiler_params=pltpu.CompilerParams(
            use_tc_tiling_on_sc=False, needs_layout_passes=False),
    )
    def k(idx_hbm, table_hbm, out_hbm, idx_v, table_v, rows0, rows1, s0, s1):
        wid = lax.axis_index("s") * nc + lax.axis_index("c")
        base = wid * per_w
        pltpu.sync_copy(table_hbm, table_v)
        pltpu.sync_copy(idx_hbm.at[pl.ds(base, per_w)], idx_v)

        lane = lax.iota(jnp.int32, L)
        colmod = [(lane + c) & (L - 1) for c in range(L)]

        BATCH = 8

        def compute_group(g, rows_ref):
            @plsc.parallel_loop(0, R // L)
            def blk(i):
                bins = idx_v[pl.ds(g * R + i * L, L)]
                rowv = i * L + lane
                for cb in range(0, EMBED_DIM, L):
                    for c0 in range(0, L, BATCH):
                        colvs = [colmod[c0 + c] + cb for c in range(BATCH)]
                        vs = [plsc.load_gather(table_v, [bins, cv])
                              for cv in colvs]
                        for cv, v in zip(colvs, vs):
                            plsc.store_scatter(rows_ref, [rowv, cv], v)

        def fire_store(g, rows_ref, sem):
            return pltpu.async_copy(
                rows_ref, out_hbm.at[pl.ds(base + g * R, R)], sem)

        def wait_store(g, rows_ref, sem):
            pltpu.make_async_copy(
                rows_ref, out_hbm.at[pl.ds(base + g * R, R)], sem).wait()

        compute_group(0, rows0)
        fire_store(0, rows0, s0)
        compute_group(1, rows1)
        fire_store(1, rows1, s1)

        def body(gh, carry):
            g0 = gh * 2
            wait_store(g0 - 2, rows0, s0)
            compute_group(g0, rows0)
            fire_store(g0, rows0, s0)
            wait_store(g0 - 1, rows1, s1)
            compute_group(g0 + 1, rows1)
            fire_store(g0 + 1, rows1, s1)
            return carry

        lax.fori_loop(1, n_groups // 2, body, 0)
        wait_store(n_groups - 2, rows0, s0)
        wait_store(n_groups - 1, rows1, s1)

    return k


def kernel(velocity_bins, table):
    b, s = velocity_bins.shape
    n = b * s
    idx = velocity_bins.astype(jnp.int32).reshape(n)
    out = _sc_lookup(n)(idx, table)
    return out.reshape(b, s, EMBED_DIM)
